# baseline (speedup 1.0000x reference)
import functools

import jax
import jax.numpy as jnp
from jax import lax
from jax.experimental import pallas as pl
from jax.experimental.pallas import tpu as pltpu
from jax.experimental.pallas import tpu_sc as plsc

N = 10000
E = 320000
D_NODE = 128
D_EDGE = 16
D_MSG = 128
D_OUT = 128
D_EOUT = 16

NSC = 2
NTILE = 16
CH = 128
EB = 32

NP = 10112
EP = 327680
EPT = EP // NTILE
NCH1 = EPT // CH
EPW = EP // (NSC * NTILE)
NCH2 = EPW // CH
RPT = NP // NTILE

_mesh = plsc.VectorSubcoreMesh(core_axis_name="c", subcore_axis_name="s")


def _mm_bias(x, w, b, blk):
    m, k = x.shape
    d = w.shape[1]
    return pl.pallas_call(
        lambda x_ref, w_ref, b_ref, o_ref: o_ref.__setitem__(
            ..., jnp.dot(x_ref[...], w_ref[...],
                         preferred_element_type=jnp.float32) + b_ref[...]),
        grid=(m // blk,),
        in_specs=[
            pl.BlockSpec((blk, k), lambda i: (i, 0)),
            pl.BlockSpec((k, d), lambda i: (0, 0)),
            pl.BlockSpec((1, d), lambda i: (0, 0)),
        ],
        out_specs=pl.BlockSpec((blk, d), lambda i: (i, 0)),
        out_shape=jax.ShapeDtypeStruct((m, d), jnp.float32),
    )(x, w, b)


def _edge_proj(edge_emb_p, p2, w1, bw):
    blk = 4096

    def body(x_ref, p2_ref, w1_ref, bw_ref, epe_ref, ew_ref):
        x = x_ref[...]
        epe_ref[...] = jnp.dot(x, p2_ref[...], preferred_element_type=jnp.float32)
        ew_ref[...] = jnp.dot(x, w1_ref[...],
                              preferred_element_type=jnp.float32) + bw_ref[...]

    return pl.pallas_call(
        body,
        grid=(EP // blk,),
        in_specs=[
            pl.BlockSpec((blk, D_EDGE), lambda i: (i, 0)),
            pl.BlockSpec((D_EDGE, D_MSG), lambda i: (0, 0)),
            pl.BlockSpec((D_EDGE, D_EOUT), lambda i: (0, 0)),
            pl.BlockSpec((1, D_EOUT), lambda i: (0, 0)),
        ],
        out_specs=[
            pl.BlockSpec((blk, D_MSG), lambda i: (i, 0)),
            pl.BlockSpec((blk, D_EOUT), lambda i: (i, 0)),
        ],
        out_shape=[
            jax.ShapeDtypeStruct((EP, D_MSG), jnp.float32),
            jax.ShapeDtypeStruct((EP, D_EOUT), jnp.float32),
        ],
    )(edge_emb_p, p2, w1, bw)


def _node_block(emb_p, sums, cnt, q, bq, w_out):
    blk = 128

    def body(emb_ref, sums_ref, cnt_ref, q_ref, bq_ref, w_ref, no_ref, a_ref):
        cnt = jnp.maximum(jnp.sum(cnt_ref[...], axis=0), 1.0)[:, None]
        msg = sums_ref[...] / cnt
        x = jnp.concatenate([emb_ref[...], msg], axis=1)
        no = jnp.dot(x, q_ref[...], preferred_element_type=jnp.float32) + bq_ref[...]
        no_ref[...] = no
        a_ref[...] = jnp.dot(no, w_ref[...], preferred_element_type=jnp.float32)

    return pl.pallas_call(
        body,
        grid=(NP // blk,),
        in_specs=[
            pl.BlockSpec((blk, D_NODE), lambda i: (i, 0)),
            pl.BlockSpec((blk, D_MSG), lambda i: (i, 0)),
            pl.BlockSpec((NTILE, blk), lambda i: (0, i)),
            pl.BlockSpec((D_NODE + D_MSG, D_OUT), lambda i: (0, 0)),
            pl.BlockSpec((1, D_OUT), lambda i: (0, 0)),
            pl.BlockSpec((D_OUT, D_NODE), lambda i: (0, 0)),
        ],
        out_specs=[
            pl.BlockSpec((blk, D_OUT), lambda i: (i, 0)),
            pl.BlockSpec((blk, D_NODE), lambda i: (i, 0)),
        ],
        out_shape=[
            jax.ShapeDtypeStruct((NP, D_OUT), jnp.float32),
            jax.ShapeDtypeStruct((NP, D_NODE), jnp.float32),
        ],
    )(emb_p, sums, cnt, q, bq, w_out)


def _sc1_body(tbl_h, epe_h, idx_h, sums_h, cnt_h,
              idx_v, rows_v, epe_v, acc, cnt_v, sems):
    c = lax.axis_index("c")
    s = lax.axis_index("s")

    def _zero(r, carry):
        for cc in range(D_MSG // 16):
            rows_v[0, r, pl.ds(cc * 16, 16)] = jnp.zeros((16,), jnp.float32)
        return carry
    lax.fori_loop(0, CH, _zero, 0)

    def _zcnt(j, carry):
        cnt_v[pl.ds(j * 16, 16)] = jnp.zeros((16,), jnp.float32)
        return carry
    lax.fori_loop(0, NP // 16, _zcnt, 0)

    base_r = s * RPT
    for j in range(4):
        pltpu.sync_copy(rows_v.at[0], acc.at[pl.ds(base_r + j * CH, CH)])
    pltpu.sync_copy(rows_v.at[0].at[pl.ds(0, RPT - 4 * CH)],
                    acc.at[pl.ds(base_r + 4 * CH, RPT - 4 * CH)])
    plsc.subcore_barrier()

    def pair(gq, carry):
        for b in range(2):
            kq = 2 * gq + b
            be = s * EPT + kq * CH
            bg = c * EP + be

            @pl.when(kq >= 2)
            def _():
                pltpu.make_async_copy(rows_v.at[b], acc.at[idx_v.at[2 + b]],
                                      sems.at[1 + b]).wait()

            pltpu.sync_copy(idx_h.at[pl.ds(bg, CH)], idx_v.at[b])
            pltpu.sync_copy(idx_h.at[pl.ds(2 * EP + bg, CH)],
                            idx_v.at[2 + b])
            gth = pltpu.async_copy(tbl_h.at[idx_v.at[b]], rows_v.at[b],
                                   sems.at[0])

            def _ld(i):
                pltpu.sync_copy(epe_h.at[pl.ds(be + i * EB, EB)], epe_v)

            def _compute(i, _b=b):
                def row(r, rc):
                    for cc in range(D_MSG // 16):
                        csl = pl.ds(cc * 16, 16)
                        rows_v[_b, i * EB + r, csl] = jnp.maximum(
                            rows_v[_b, i * EB + r, csl]
                            + epe_v[r, csl], 0.0)
                    return rc
                lax.fori_loop(0, EB, row, 0)

            _ld(0)
            gth.wait()
            _compute(0)
            _ld(1)
            _compute(1)
            _ld(2)
            _compute(2)
            _ld(3)
            _compute(3)

            pltpu.async_copy(rows_v.at[b], acc.at[idx_v.at[2 + b]],
                             sems.at[1 + b], add=True)
            for i in range(CH // 16):
                idx16 = idx_v[2 + b, pl.ds(i * 16, 16)]
                plsc.addupdate_scatter(cnt_v, [idx16],
                                       jnp.full((16,), 1.0, jnp.float32))
        return carry
    lax.fori_loop(0, NCH1 // 2, pair, 0)

    for b in range(2):
        pltpu.make_async_copy(rows_v.at[b], acc.at[idx_v.at[2 + b]],
                              sems.at[1 + b]).wait()

    plsc.subcore_barrier()
    for j in range(4):
        osl = pl.ds(base_r + j * CH, CH)
        pltpu.sync_copy(acc.at[osl], rows_v.at[0])
        pltpu.sync_copy(rows_v.at[0],
                        sums_h.at[pl.ds(c * NP + base_r + j * CH, CH)])
    vsl = pl.ds(0, RPT - 4 * CH)
    osl = pl.ds(base_r + 4 * CH, RPT - 4 * CH)
    hsl = pl.ds(c * NP + base_r + 4 * CH, RPT - 4 * CH)
    pltpu.sync_copy(acc.at[osl], rows_v.at[0].at[vsl])
    pltpu.sync_copy(rows_v.at[0].at[vsl], sums_h.at[hsl])
    pltpu.sync_copy(cnt_v, cnt_h.at[pl.ds((c * NTILE + s) * NP, NP)])


_sc1 = functools.partial(
    pl.kernel,
    out_type=[
        jax.ShapeDtypeStruct((2 * NP, D_MSG), jnp.float32),
        jax.ShapeDtypeStruct((2 * NTILE * NP,), jnp.float32),
    ],
    mesh=_mesh,
    compiler_params=pltpu.CompilerParams(needs_layout_passes=False),
    scratch_types=[
        pltpu.VMEM((4, CH), jnp.int32),
        pltpu.VMEM((2, CH, D_MSG), jnp.float32),
        pltpu.VMEM((EB, D_MSG), jnp.float32),
        pltpu.VMEM_SHARED((NP, D_MSG), jnp.float32),
        pltpu.VMEM((NP,), jnp.float32),
        pltpu.SemaphoreType.DMA((3,)),
    ],
)(_sc1_body)


def _sc2_body(a_h, bv_h, ewf_h, idx_h, eof_h,
              idx_v, arows_v, brows_v, eo_v, sems):
    c = lax.axis_index("c")
    s = lax.axis_index("s")
    base0 = (s * NSC + c) * EPW

    def chunk(kq, b):
        bq = base0 + kq * CH
        osl = pl.ds(bq * D_EOUT, CH * D_EOUT)

        @pl.when(jnp.logical_and(kq >= 2, bq - 2 * CH < E))
        def _():
            pltpu.make_async_copy(eo_v.at[b], eof_h.at[pl.ds(0, CH * D_EOUT)],
                                  sems.at[1 + b]).wait()

        pltpu.sync_copy(idx_h.at[pl.ds(bq, CH)], idx_v.at[b])
        pltpu.sync_copy(idx_h.at[pl.ds(EP + bq, CH)], idx_v.at[2 + b])
        g1 = pltpu.async_copy(a_h.at[idx_v.at[b]], arows_v.at[b], sems.at[0])
        g2 = pltpu.async_copy(bv_h.at[idx_v.at[2 + b]], brows_v.at[b],
                              sems.at[0])
        pltpu.sync_copy(ewf_h.at[pl.ds(bq * D_EOUT, CH * D_EOUT)], eo_v.at[b])
        g1.wait()
        g2.wait()

        def row(r, rc, _b=b):
            esl = pl.ds(r * D_EOUT, D_EOUT)
            csl = pl.ds(0, D_EOUT)
            eo_v[_b, esl] = (eo_v[_b, esl] + arows_v[_b, r, csl]
                             + brows_v[_b, r, csl])
            return rc
        lax.fori_loop(0, CH, row, 0)

        @pl.when(bq < E)
        def _():
            pltpu.async_copy(eo_v.at[b], eof_h.at[osl], sems.at[1 + b])

    def pair(gq, carry):
        for b in range(2):
            chunk(2 * gq + b, b)
        return carry
    lax.fori_loop(0, NCH2 // 2, pair, 0)
    for b in range(2):
        @pl.when(base0 + (NCH2 - 2 + b) * CH < E)
        def _():
            pltpu.make_async_copy(
                eo_v.at[b], eof_h.at[pl.ds(0, CH * D_EOUT)],
                sems.at[1 + b]).wait()


_sc2 = functools.partial(
    pl.kernel,
    out_type=jax.ShapeDtypeStruct((E * D_EOUT,), jnp.float32),
    mesh=_mesh,
    compiler_params=pltpu.CompilerParams(needs_layout_passes=False),
    scratch_types=[
        pltpu.VMEM((4, CH), jnp.int32),
        pltpu.VMEM((2, CH, D_NODE), jnp.float32),
        pltpu.VMEM((2, CH, D_NODE), jnp.float32),
        pltpu.VMEM((2, CH * D_EOUT), jnp.float32),
        pltpu.SemaphoreType.DMA((3,)),
    ],
)(_sc2_body)


def kernel(node_emb, edge_emb, feature_emb, edge_index, P, Q, W, b_P, b_Q, b_W):
    src = edge_index[0]
    dst = edge_index[1]

    p1, p2 = P[:D_NODE], P[D_NODE:]
    w1, w2, w3 = W[:D_EDGE], W[D_EDGE:D_EDGE + D_OUT], W[D_EDGE + D_OUT:]

    pad0 = jnp.zeros((EP - E,), jnp.int32)
    padn = jnp.full((EP - E,), N, jnp.int32)
    src_g = jnp.concatenate([src, pad0])
    dst_g = jnp.concatenate([dst, pad0])
    src_s = jnp.concatenate([src, padn])
    dst_s = jnp.concatenate([dst, padn])
    edge_emb_p = jnp.pad(edge_emb, ((0, EP - E), (0, 0)))

    g = _mm_bias(feature_emb, p1, b_P.reshape(1, -1), 2000)
    gn = _mm_bias(node_emb, p1, b_P.reshape(1, -1), 2000)
    epe, ew = _edge_proj(edge_emb_p, p2, w1, b_W.reshape(1, -1))

    tbl = jnp.concatenate([g, gn], axis=0)
    idx1 = jnp.concatenate([dst_g, src_g + N, src_s, dst_s])
    sums, cnts = _sc1(tbl, epe, idx1)
    ssrc, sdst = sums[:NP], sums[NP:]
    cnts = cnts.reshape(2, NTILE, NP)
    csrc, cdst = cnts[0], cnts[1]

    node_emb_p = jnp.pad(node_emb, ((0, NP - N), (0, 0)))
    feature_emb_p = jnp.pad(feature_emb, ((0, NP - N), (0, 0)))
    w2p = jnp.pad(w2, ((0, 0), (0, D_NODE - D_EOUT)))
    w3p = jnp.pad(w3, ((0, 0), (0, D_NODE - D_EOUT)))
    node_out, a = _node_block(node_emb_p, ssrc, csrc, Q, b_Q.reshape(1, -1),
                              w2p)
    feat_out, bv = _node_block(feature_emb_p, sdst, cdst, Q,
                               b_Q.reshape(1, -1), w3p)

    idx2 = jnp.concatenate([src_g, dst_g])
    eof = _sc2(a, bv, ew.reshape(-1), idx2)
    return node_out[:N], eof.reshape(E, D_EOUT), feat_out[:N]


# speedup vs baseline: 1.1381x; 1.1381x over previous
import functools

import jax
import jax.numpy as jnp
from jax import lax
from jax.experimental import pallas as pl
from jax.experimental.pallas import tpu as pltpu
from jax.experimental.pallas import tpu_sc as plsc

N = 10000
E = 320000
D_NODE = 128
D_EDGE = 16
D_MSG = 128
D_OUT = 128
D_EOUT = 16

NSC = 2
NTILE = 16
CH = 128
EB = 32

NP = 10112
EP = 323584
EPT = EP // NTILE
NCH1 = EPT // CH
EPW = EP // (NSC * NTILE)
NCH2 = EPW // CH
RPT = NP // NTILE

_mesh = plsc.VectorSubcoreMesh(core_axis_name="c", subcore_axis_name="s")


def _mm_bias(x, w, b, blk):
    m, k = x.shape
    d = w.shape[1]
    return pl.pallas_call(
        lambda x_ref, w_ref, b_ref, o_ref: o_ref.__setitem__(
            ..., jnp.dot(x_ref[...], w_ref[...],
                         preferred_element_type=jnp.float32) + b_ref[...]),
        grid=(m // blk,),
        in_specs=[
            pl.BlockSpec((blk, k), lambda i: (i, 0)),
            pl.BlockSpec((k, d), lambda i: (0, 0)),
            pl.BlockSpec((1, d), lambda i: (0, 0)),
        ],
        out_specs=pl.BlockSpec((blk, d), lambda i: (i, 0)),
        out_shape=jax.ShapeDtypeStruct((m, d), jnp.float32),
    )(x, w, b)


def _edge_proj(edge_emb_p, p2, w1, bw):
    blk = 4096

    def body(x_ref, p2_ref, w1_ref, bw_ref, epe_ref, ew_ref):
        x = x_ref[...]
        epe_ref[...] = jnp.dot(x, p2_ref[...], preferred_element_type=jnp.float32)
        ew_ref[...] = jnp.dot(x, w1_ref[...],
                              preferred_element_type=jnp.float32) + bw_ref[...]

    return pl.pallas_call(
        body,
        grid=(EP // blk,),
        in_specs=[
            pl.BlockSpec((blk, D_EDGE), lambda i: (i, 0)),
            pl.BlockSpec((D_EDGE, D_MSG), lambda i: (0, 0)),
            pl.BlockSpec((D_EDGE, D_EOUT), lambda i: (0, 0)),
            pl.BlockSpec((1, D_EOUT), lambda i: (0, 0)),
        ],
        out_specs=[
            pl.BlockSpec((blk, D_MSG), lambda i: (i, 0)),
            pl.BlockSpec((blk, D_EOUT), lambda i: (i, 0)),
        ],
        out_shape=[
            jax.ShapeDtypeStruct((EP, D_MSG), jnp.float32),
            jax.ShapeDtypeStruct((EP, D_EOUT), jnp.float32),
        ],
    )(edge_emb_p, p2, w1, bw)


def _node_block(emb_p, sums, cnt, q, bq, w_out, v):
    blk = 128

    def body(emb_ref, sums_ref, cnt_ref, q_ref, bq_ref, w_ref, no_ref, a_ref):
        cnt = jnp.maximum(jnp.sum(cnt_ref[...], axis=0), 1.0)[:, None]
        msg = sums_ref[...] / cnt
        x = jnp.concatenate([emb_ref[...], msg], axis=1)
        no = jnp.dot(x, q_ref[...], preferred_element_type=jnp.float32) + bq_ref[...]
        no_ref[...] = no
        a_ref[...] = jnp.dot(no, w_ref[...], preferred_element_type=jnp.float32)

    return pl.pallas_call(
        body,
        grid=(NP // blk,),
        in_specs=[
            pl.BlockSpec((blk, D_NODE), lambda i: (i, 0)),
            pl.BlockSpec((blk, D_MSG), lambda i, _v=v: (_v * (NP // blk) + i, 0)),
            pl.BlockSpec((NTILE, blk), lambda i, _v=v: (_v, i)),
            pl.BlockSpec((D_NODE + D_MSG, D_OUT), lambda i: (0, 0)),
            pl.BlockSpec((1, D_OUT), lambda i: (0, 0)),
            pl.BlockSpec((D_OUT, D_NODE), lambda i: (0, 0)),
        ],
        out_specs=[
            pl.BlockSpec((blk, D_OUT), lambda i: (i, 0)),
            pl.BlockSpec((blk, D_NODE), lambda i: (i, 0)),
        ],
        out_shape=[
            jax.ShapeDtypeStruct((NP, D_OUT), jnp.float32),
            jax.ShapeDtypeStruct((NP, D_NODE), jnp.float32),
        ],
    )(emb_p, sums, cnt, q, bq, w_out)


def _sc1_body(tbl_h, epe_h, idx_h, sums_h, cnt_h,
              idx_v, rows_v, epe_v, acc, cnt_v, sems):
    c = lax.axis_index("c")
    s = lax.axis_index("s")

    def _zero(r, carry):
        for cc in range(D_MSG // 16):
            rows_v[0, r, pl.ds(cc * 16, 16)] = jnp.zeros((16,), jnp.float32)
        return carry
    lax.fori_loop(0, CH, _zero, 0)

    def _zcnt(j, carry):
        cnt_v[pl.ds(j * 16, 16)] = jnp.zeros((16,), jnp.float32)
        return carry
    lax.fori_loop(0, NP // 16, _zcnt, 0)

    base_r = s * RPT
    for j in range(4):
        pltpu.sync_copy(rows_v.at[0], acc.at[pl.ds(base_r + j * CH, CH)])
    pltpu.sync_copy(rows_v.at[0].at[pl.ds(0, RPT - 4 * CH)],
                    acc.at[pl.ds(base_r + 4 * CH, RPT - 4 * CH)])
    plsc.subcore_barrier()

    def pair(gq, carry):
        for b in range(2):
            kq = 2 * gq + b
            be = s * EPT + kq * CH
            bg = c * EP + be

            @pl.when(kq >= 2)
            def _():
                pltpu.make_async_copy(rows_v.at[b], acc.at[idx_v.at[2 + b]],
                                      sems.at[1 + b]).wait()

            pltpu.sync_copy(idx_h.at[pl.ds(bg, CH)], idx_v.at[b])
            pltpu.sync_copy(idx_h.at[pl.ds(2 * EP + bg, CH)],
                            idx_v.at[2 + b])
            gth = pltpu.async_copy(tbl_h.at[idx_v.at[b]], rows_v.at[b],
                                   sems.at[0])

            def _ld(i):
                pltpu.sync_copy(epe_h.at[pl.ds(be + i * EB, EB)], epe_v)

            def _compute(i, _b=b):
                def row(r, rc):
                    for cc in range(D_MSG // 16):
                        csl = pl.ds(cc * 16, 16)
                        rows_v[_b, i * EB + r, csl] = jnp.maximum(
                            rows_v[_b, i * EB + r, csl]
                            + epe_v[r, csl], 0.0)
                    return rc
                lax.fori_loop(0, EB, row, 0)

            _ld(0)
            gth.wait()
            _compute(0)
            _ld(1)
            _compute(1)
            _ld(2)
            _compute(2)
            _ld(3)
            _compute(3)

            pltpu.async_copy(rows_v.at[b], acc.at[idx_v.at[2 + b]],
                             sems.at[1 + b], add=True)
            for i in range(CH // 16):
                idx16 = idx_v[2 + b, pl.ds(i * 16, 16)]
                plsc.addupdate_scatter(cnt_v, [idx16],
                                       jnp.full((16,), 1.0, jnp.float32))
        return carry
    lax.fori_loop(0, NCH1 // 2, pair, 0)

    for b in range(2):
        pltpu.make_async_copy(rows_v.at[b], acc.at[idx_v.at[2 + b]],
                              sems.at[1 + b]).wait()

    plsc.subcore_barrier()
    for j in range(4):
        osl = pl.ds(base_r + j * CH, CH)
        pltpu.sync_copy(acc.at[osl], rows_v.at[0])
        pltpu.sync_copy(rows_v.at[0],
                        sums_h.at[pl.ds(c * NP + base_r + j * CH, CH)])
    vsl = pl.ds(0, RPT - 4 * CH)
    osl = pl.ds(base_r + 4 * CH, RPT - 4 * CH)
    hsl = pl.ds(c * NP + base_r + 4 * CH, RPT - 4 * CH)
    pltpu.sync_copy(acc.at[osl], rows_v.at[0].at[vsl])
    pltpu.sync_copy(rows_v.at[0].at[vsl], sums_h.at[hsl])
    pltpu.sync_copy(cnt_v, cnt_h.at[pl.ds((c * NTILE + s) * NP, NP)])


_sc1 = functools.partial(
    pl.kernel,
    out_type=[
        jax.ShapeDtypeStruct((2 * NP, D_MSG), jnp.float32),
        jax.ShapeDtypeStruct((2 * NTILE * NP,), jnp.float32),
    ],
    mesh=_mesh,
    compiler_params=pltpu.CompilerParams(needs_layout_passes=False),
    scratch_types=[
        pltpu.VMEM((4, CH), jnp.int32),
        pltpu.VMEM((2, CH, D_MSG), jnp.float32),
        pltpu.VMEM((EB, D_MSG), jnp.float32),
        pltpu.VMEM_SHARED((NP, D_MSG), jnp.float32),
        pltpu.VMEM((NP,), jnp.float32),
        pltpu.SemaphoreType.DMA((3,)),
    ],
)(_sc1_body)


def _sc2_body(a_h, bv_h, ewf_h, idx_h, eof_h,
              idx_v, arows_v, brows_v, eo_v, sems):
    c = lax.axis_index("c")
    s = lax.axis_index("s")
    base0 = (s * NSC + c) * EPW

    def chunk(kq, b):
        bq = base0 + kq * CH
        osl = pl.ds(bq * D_EOUT, CH * D_EOUT)

        @pl.when(jnp.logical_and(kq >= 2, bq - 2 * CH < E))
        def _():
            pltpu.make_async_copy(eo_v.at[b], eof_h.at[pl.ds(0, CH * D_EOUT)],
                                  sems.at[1 + b]).wait()

        pltpu.sync_copy(idx_h.at[pl.ds(bq, CH)], idx_v.at[b])
        pltpu.sync_copy(idx_h.at[pl.ds(EP + bq, CH)], idx_v.at[2 + b])
        g1 = pltpu.async_copy(a_h.at[idx_v.at[b]], arows_v.at[b], sems.at[0])
        g2 = pltpu.async_copy(bv_h.at[idx_v.at[2 + b]], brows_v.at[b],
                              sems.at[0])
        pltpu.sync_copy(ewf_h.at[pl.ds(bq * D_EOUT, CH * D_EOUT)], eo_v.at[b])
        g1.wait()
        g2.wait()

        def row(r, rc, _b=b):
            esl = pl.ds(r * D_EOUT, D_EOUT)
            csl = pl.ds(0, D_EOUT)
            eo_v[_b, esl] = (eo_v[_b, esl] + arows_v[_b, r, csl]
                             + brows_v[_b, r, csl])
            return rc
        lax.fori_loop(0, CH, row, 0)

        @pl.when(bq < E)
        def _():
            pltpu.async_copy(eo_v.at[b], eof_h.at[osl], sems.at[1 + b])

    def pair(gq, carry):
        for b in range(2):
            chunk(2 * gq + b, b)
        return carry
    lax.fori_loop(0, NCH2 // 2, pair, 0)
    chunk(NCH2 - 1, 0)
    for b in range(2):
        @pl.when(base0 + (NCH2 - 1 - b) * CH < E)
        def _():
            pltpu.make_async_copy(
                eo_v.at[b], eof_h.at[pl.ds(0, CH * D_EOUT)],
                sems.at[1 + b]).wait()


_sc2 = functools.partial(
    pl.kernel,
    out_type=jax.ShapeDtypeStruct((E * D_EOUT,), jnp.float32),
    mesh=_mesh,
    compiler_params=pltpu.CompilerParams(needs_layout_passes=False),
    scratch_types=[
        pltpu.VMEM((4, CH), jnp.int32),
        pltpu.VMEM((2, CH, D_NODE), jnp.float32),
        pltpu.VMEM((2, CH, D_NODE), jnp.float32),
        pltpu.VMEM((2, CH * D_EOUT), jnp.float32),
        pltpu.SemaphoreType.DMA((3,)),
    ],
)(_sc2_body)


def kernel(node_emb, edge_emb, feature_emb, edge_index, P, Q, W, b_P, b_Q, b_W):
    src = edge_index[0]
    dst = edge_index[1]

    p1, p2 = P[:D_NODE], P[D_NODE:]
    w1, w2, w3 = W[:D_EDGE], W[D_EDGE:D_EDGE + D_OUT], W[D_EDGE + D_OUT:]

    pad0 = jnp.zeros((EP - E,), jnp.int32)
    padn = jnp.full((EP - E,), N, jnp.int32)
    src_g = jnp.concatenate([src, pad0])
    dst_g = jnp.concatenate([dst, pad0])
    src_s = jnp.concatenate([src, padn])
    dst_s = jnp.concatenate([dst, padn])
    edge_emb_p = jnp.pad(edge_emb, ((0, EP - E), (0, 0)))

    g = _mm_bias(feature_emb, p1, b_P.reshape(1, -1), 2000)
    gn = _mm_bias(node_emb, p1, b_P.reshape(1, -1), 2000)
    epe, ew = _edge_proj(edge_emb_p, p2, w1, b_W.reshape(1, -1))

    tbl = jnp.concatenate([g, gn], axis=0)
    idx1 = jnp.concatenate([dst_g, src_g + N, src_s, dst_s])
    sums, cnts = _sc1(tbl, epe, idx1)
    cnts2 = cnts.reshape(2 * NTILE, NP)

    node_emb_p = jnp.pad(node_emb, ((0, NP - N), (0, 0)))
    feature_emb_p = jnp.pad(feature_emb, ((0, NP - N), (0, 0)))
    w2p = jnp.pad(w2, ((0, 0), (0, D_NODE - D_EOUT)))
    w3p = jnp.pad(w3, ((0, 0), (0, D_NODE - D_EOUT)))
    node_out, a = _node_block(node_emb_p, sums, cnts2, Q, b_Q.reshape(1, -1),
                              w2p, 0)
    feat_out, bv = _node_block(feature_emb_p, sums, cnts2, Q,
                               b_Q.reshape(1, -1), w3p, 1)

    idx2 = jnp.concatenate([src_g, dst_g])
    eof = _sc2(a, bv, ew.reshape(-1), idx2)
    return node_out[:N], eof.reshape(E, D_EOUT), feat_out[:N]


# speedup vs baseline: 1.1502x; 1.0106x over previous
import functools

import jax
import jax.numpy as jnp
from jax import lax
from jax.experimental import pallas as pl
from jax.experimental.pallas import tpu as pltpu
from jax.experimental.pallas import tpu_sc as plsc

N = 10000
E = 320000
D_NODE = 128
D_EDGE = 16
D_MSG = 128
D_OUT = 128
D_EOUT = 16

NSC = 2
NTILE = 16
CH = 128
EB = 32

NP = 10112
EP = 323584
EPT = EP // NTILE
NCH1 = EPT // CH
EPW = EP // (NSC * NTILE)
NCH2 = EPW // CH
RPT = NP // NTILE

_mesh = plsc.VectorSubcoreMesh(core_axis_name="c", subcore_axis_name="s")


def _mm_bias(x, w, b, blk):
    m, k = x.shape
    d = w.shape[1]
    return pl.pallas_call(
        lambda x_ref, w_ref, b_ref, o_ref: o_ref.__setitem__(
            ..., jnp.dot(x_ref[...], w_ref[...],
                         preferred_element_type=jnp.float32) + b_ref[...]),
        grid=(m // blk,),
        in_specs=[
            pl.BlockSpec((blk, k), lambda i: (i, 0)),
            pl.BlockSpec((k, d), lambda i: (0, 0)),
            pl.BlockSpec((1, d), lambda i: (0, 0)),
        ],
        out_specs=pl.BlockSpec((blk, d), lambda i: (i, 0)),
        out_shape=jax.ShapeDtypeStruct((m, d), jnp.float32),
    )(x, w, b)


def _edge_proj(edge_emb_p, p2, w1, bw):
    blk = 4096

    def body(x_ref, p2_ref, w1_ref, bw_ref, epe_ref, ew_ref):
        x = x_ref[...]
        epe_ref[...] = jnp.dot(x, p2_ref[...], preferred_element_type=jnp.float32)
        ew_ref[...] = jnp.dot(x, w1_ref[...],
                              preferred_element_type=jnp.float32) + bw_ref[...]

    return pl.pallas_call(
        body,
        grid=(EP // blk,),
        in_specs=[
            pl.BlockSpec((blk, D_EDGE), lambda i: (i, 0)),
            pl.BlockSpec((D_EDGE, D_MSG), lambda i: (0, 0)),
            pl.BlockSpec((D_EDGE, D_EOUT), lambda i: (0, 0)),
            pl.BlockSpec((1, D_EOUT), lambda i: (0, 0)),
        ],
        out_specs=[
            pl.BlockSpec((blk, D_MSG), lambda i: (i, 0)),
            pl.BlockSpec((blk, D_EOUT), lambda i: (i, 0)),
        ],
        out_shape=[
            jax.ShapeDtypeStruct((EP, D_MSG), jnp.float32),
            jax.ShapeDtypeStruct((EP, D_EOUT), jnp.float32),
        ],
    )(edge_emb_p, p2, w1, bw)


def _node_block(emb_p, sums, cnt, q, bq, w_out, v):
    blk = 128

    def body(emb_ref, sums_ref, cnt_ref, q_ref, bq_ref, w_ref, no_ref, a_ref):
        cnt = jnp.maximum(jnp.sum(cnt_ref[...], axis=0), 1.0)[:, None]
        msg = sums_ref[...] / cnt
        x = jnp.concatenate([emb_ref[...], msg], axis=1)
        no = jnp.dot(x, q_ref[...], preferred_element_type=jnp.float32) + bq_ref[...]
        no_ref[...] = no
        a_ref[...] = jnp.dot(no, w_ref[...], preferred_element_type=jnp.float32)

    return pl.pallas_call(
        body,
        grid=(NP // blk,),
        in_specs=[
            pl.BlockSpec((blk, D_NODE), lambda i: (i, 0)),
            pl.BlockSpec((blk, D_MSG), lambda i, _v=v: (_v * (NP // blk) + i, 0)),
            pl.BlockSpec((NTILE, blk), lambda i, _v=v: (_v, i)),
            pl.BlockSpec((D_NODE + D_MSG, D_OUT), lambda i: (0, 0)),
            pl.BlockSpec((1, D_OUT), lambda i: (0, 0)),
            pl.BlockSpec((D_OUT, D_NODE), lambda i: (0, 0)),
        ],
        out_specs=[
            pl.BlockSpec((blk, D_OUT), lambda i: (i, 0)),
            pl.BlockSpec((blk, D_NODE), lambda i: (i, 0)),
        ],
        out_shape=[
            jax.ShapeDtypeStruct((NP, D_OUT), jnp.float32),
            jax.ShapeDtypeStruct((NP, D_NODE), jnp.float32),
        ],
    )(emb_p, sums, cnt, q, bq, w_out)


def _sc1_body(tbl_h, epe_h, idx_h, sums_h, cnt_h,
              idx_v, gidx_v, rows_v, epe_v, acc, cnt_v, sems):
    c = lax.axis_index("c")
    s = lax.axis_index("s")

    def _zero(r, carry):
        for cc in range(D_MSG // 16):
            rows_v[0, r, pl.ds(cc * 16, 16)] = jnp.zeros((16,), jnp.float32)
        return carry
    lax.fori_loop(0, CH, _zero, 0)

    def _zcnt(j, carry):
        cnt_v[pl.ds(j * 16, 16)] = jnp.zeros((16,), jnp.float32)
        return carry
    lax.fori_loop(0, NP // 16, _zcnt, 0)

    base_r = s * RPT
    for j in range(4):
        pltpu.sync_copy(rows_v.at[0], acc.at[pl.ds(base_r + j * CH, CH)])
    pltpu.sync_copy(rows_v.at[0].at[pl.ds(0, RPT - 4 * CH)],
                    acc.at[pl.ds(base_r + 4 * CH, RPT - 4 * CH)])
    plsc.subcore_barrier()

    def pair(gq, carry):
        pltpu.sync_copy(
            idx_h.at[pl.ds(c * EP + s * EPT + 2 * gq * CH, 2 * CH)], gidx_v)
        for b in range(2):
            kq = 2 * gq + b
            be = s * EPT + kq * CH
            bg = c * EP + be

            @pl.when(kq >= 2)
            def _():
                pltpu.make_async_copy(rows_v.at[b], acc.at[idx_v.at[2 + b]],
                                      sems.at[1 + b]).wait()

            pltpu.sync_copy(idx_h.at[pl.ds(2 * EP + bg, CH)],
                            idx_v.at[2 + b])
            gth = pltpu.async_copy(tbl_h.at[gidx_v.at[pl.ds(b * CH, CH)]],
                                   rows_v.at[b], sems.at[0])

            def _ld(i):
                pltpu.sync_copy(epe_h.at[pl.ds(be + i * EB, EB)], epe_v)

            def _compute(i, _b=b):
                def row(r, rc):
                    for cc in range(D_MSG // 16):
                        csl = pl.ds(cc * 16, 16)
                        rows_v[_b, i * EB + r, csl] = jnp.maximum(
                            rows_v[_b, i * EB + r, csl]
                            + epe_v[r, csl], 0.0)
                    return rc
                lax.fori_loop(0, EB, row, 0)

            _ld(0)
            gth.wait()
            _compute(0)
            _ld(1)
            _compute(1)
            _ld(2)
            _compute(2)
            _ld(3)
            _compute(3)

            pltpu.async_copy(rows_v.at[b], acc.at[idx_v.at[2 + b]],
                             sems.at[1 + b], add=True)
            for i in range(CH // 16):
                idx16 = idx_v[2 + b, pl.ds(i * 16, 16)]
                plsc.addupdate_scatter(cnt_v, [idx16],
                                       jnp.full((16,), 1.0, jnp.float32))
        return carry
    lax.fori_loop(0, NCH1 // 2, pair, 0)

    for b in range(2):
        pltpu.make_async_copy(rows_v.at[b], acc.at[idx_v.at[2 + b]],
                              sems.at[1 + b]).wait()

    plsc.subcore_barrier()
    for j in range(4):
        osl = pl.ds(base_r + j * CH, CH)
        pltpu.sync_copy(acc.at[osl], rows_v.at[0])
        pltpu.sync_copy(rows_v.at[0],
                        sums_h.at[pl.ds(c * NP + base_r + j * CH, CH)])
    vsl = pl.ds(0, RPT - 4 * CH)
    osl = pl.ds(base_r + 4 * CH, RPT - 4 * CH)
    hsl = pl.ds(c * NP + base_r + 4 * CH, RPT - 4 * CH)
    pltpu.sync_copy(acc.at[osl], rows_v.at[0].at[vsl])
    pltpu.sync_copy(rows_v.at[0].at[vsl], sums_h.at[hsl])
    pltpu.sync_copy(cnt_v, cnt_h.at[pl.ds((c * NTILE + s) * NP, NP)])


_sc1 = functools.partial(
    pl.kernel,
    out_type=[
        jax.ShapeDtypeStruct((2 * NP, D_MSG), jnp.float32),
        jax.ShapeDtypeStruct((2 * NTILE * NP,), jnp.float32),
    ],
    mesh=_mesh,
    compiler_params=pltpu.CompilerParams(needs_layout_passes=False),
    scratch_types=[
        pltpu.VMEM((4, CH), jnp.int32),
        pltpu.VMEM((2 * CH,), jnp.int32),
        pltpu.VMEM((2, CH, D_MSG), jnp.float32),
        pltpu.VMEM((EB, D_MSG), jnp.float32),
        pltpu.VMEM_SHARED((NP, D_MSG), jnp.float32),
        pltpu.VMEM((NP,), jnp.float32),
        pltpu.SemaphoreType.DMA((3,)),
    ],
)(_sc1_body)


def _sc2_body(a_h, bv_h, ewf_h, idx_h, eof_h,
              idx_v, arows_v, brows_v, eo_v, sems):
    c = lax.axis_index("c")
    s = lax.axis_index("s")
    base0 = (s * NSC + c) * EPW

    def chunk(kq, b):
        bq = base0 + kq * CH
        osl = pl.ds(bq * D_EOUT, CH * D_EOUT)

        @pl.when(jnp.logical_and(kq >= 2, bq - 2 * CH < E))
        def _():
            pltpu.make_async_copy(eo_v.at[b], eof_h.at[pl.ds(0, CH * D_EOUT)],
                                  sems.at[1 + b]).wait()

        pltpu.sync_copy(idx_h.at[pl.ds(bq, CH)], idx_v.at[b])
        pltpu.sync_copy(idx_h.at[pl.ds(EP + bq, CH)], idx_v.at[2 + b])
        g1 = pltpu.async_copy(a_h.at[idx_v.at[b]], arows_v.at[b], sems.at[0])
        g2 = pltpu.async_copy(bv_h.at[idx_v.at[2 + b]], brows_v.at[b],
                              sems.at[0])
        pltpu.sync_copy(ewf_h.at[pl.ds(bq * D_EOUT, CH * D_EOUT)], eo_v.at[b])
        g1.wait()
        g2.wait()

        def row(r, rc, _b=b):
            esl = pl.ds(r * D_EOUT, D_EOUT)
            csl = pl.ds(0, D_EOUT)
            eo_v[_b, esl] = (eo_v[_b, esl] + arows_v[_b, r, csl]
                             + brows_v[_b, r, csl])
            return rc
        lax.fori_loop(0, CH, row, 0)

        @pl.when(bq < E)
        def _():
            pltpu.async_copy(eo_v.at[b], eof_h.at[osl], sems.at[1 + b])

    def pair(gq, carry):
        for b in range(2):
            chunk(2 * gq + b, b)
        return carry
    lax.fori_loop(0, NCH2 // 2, pair, 0)
    chunk(NCH2 - 1, 0)
    for b in range(2):
        @pl.when(base0 + (NCH2 - 1 - b) * CH < E)
        def _():
            pltpu.make_async_copy(
                eo_v.at[b], eof_h.at[pl.ds(0, CH * D_EOUT)],
                sems.at[1 + b]).wait()


_sc2 = functools.partial(
    pl.kernel,
    out_type=jax.ShapeDtypeStruct((E * D_EOUT,), jnp.float32),
    mesh=_mesh,
    compiler_params=pltpu.CompilerParams(needs_layout_passes=False),
    scratch_types=[
        pltpu.VMEM((4, CH), jnp.int32),
        pltpu.VMEM((2, CH, D_NODE), jnp.float32),
        pltpu.VMEM((2, CH, D_NODE), jnp.float32),
        pltpu.VMEM((2, CH * D_EOUT), jnp.float32),
        pltpu.SemaphoreType.DMA((3,)),
    ],
)(_sc2_body)


def kernel(node_emb, edge_emb, feature_emb, edge_index, P, Q, W, b_P, b_Q, b_W):
    src = edge_index[0]
    dst = edge_index[1]

    p1, p2 = P[:D_NODE], P[D_NODE:]
    w1, w2, w3 = W[:D_EDGE], W[D_EDGE:D_EDGE + D_OUT], W[D_EDGE + D_OUT:]

    pad0 = jnp.zeros((EP - E,), jnp.int32)
    padn = jnp.full((EP - E,), N, jnp.int32)
    src_g = jnp.concatenate([src, pad0])
    dst_g = jnp.concatenate([dst, pad0])
    src_s = jnp.concatenate([src, padn])
    dst_s = jnp.concatenate([dst, padn])
    edge_emb_p = jnp.pad(edge_emb, ((0, EP - E), (0, 0)))

    g = _mm_bias(feature_emb, p1, b_P.reshape(1, -1), 2000)
    gn = _mm_bias(node_emb, p1, b_P.reshape(1, -1), 2000)
    epe, ew = _edge_proj(edge_emb_p, p2, w1, b_W.reshape(1, -1))

    tbl = jnp.concatenate([g, gn], axis=0)
    idx1 = jnp.concatenate([dst_g, src_g + N, src_s, dst_s])
    sums, cnts = _sc1(tbl, epe, idx1)
    cnts2 = cnts.reshape(2 * NTILE, NP)

    node_emb_p = jnp.pad(node_emb, ((0, NP - N), (0, 0)))
    feature_emb_p = jnp.pad(feature_emb, ((0, NP - N), (0, 0)))
    w2p = jnp.pad(w2, ((0, 0), (0, D_NODE - D_EOUT)))
    w3p = jnp.pad(w3, ((0, 0), (0, D_NODE - D_EOUT)))
    node_out, a = _node_block(node_emb_p, sums, cnts2, Q, b_Q.reshape(1, -1),
                              w2p, 0)
    feat_out, bv = _node_block(feature_emb_p, sums, cnts2, Q,
                               b_Q.reshape(1, -1), w3p, 1)

    idx2 = jnp.concatenate([src_g, dst_g])
    eof = _sc2(a, bv, ew.reshape(-1), idx2)
    return node_out[:N], eof.reshape(E, D_EOUT), feat_out[:N]


# speedup vs baseline: 1.1692x; 1.0165x over previous
import functools

import jax
import jax.numpy as jnp
from jax import lax
from jax.experimental import pallas as pl
from jax.experimental.pallas import tpu as pltpu
from jax.experimental.pallas import tpu_sc as plsc

N = 10000
E = 320000
D_NODE = 128
D_EDGE = 16
D_MSG = 128
D_OUT = 128
D_EOUT = 16

NSC = 2
NTILE = 16
CH = 128
EB = 32

NP = 10112
EP = 323584
EPT = EP // NTILE
NCH1 = EPT // CH
EPW = EP // (NSC * NTILE)
NCH2 = EPW // CH
RPT = NP // NTILE

_mesh = plsc.VectorSubcoreMesh(core_axis_name="c", subcore_axis_name="s")


def _mm_bias(x, w, b, blk):
    m, k = x.shape
    d = w.shape[1]
    return pl.pallas_call(
        lambda x_ref, w_ref, b_ref, o_ref: o_ref.__setitem__(
            ..., jnp.dot(x_ref[...], w_ref[...],
                         preferred_element_type=jnp.float32) + b_ref[...]),
        grid=(m // blk,),
        in_specs=[
            pl.BlockSpec((blk, k), lambda i: (i, 0)),
            pl.BlockSpec((k, d), lambda i: (0, 0)),
            pl.BlockSpec((1, d), lambda i: (0, 0)),
        ],
        out_specs=pl.BlockSpec((blk, d), lambda i: (i, 0)),
        out_shape=jax.ShapeDtypeStruct((m, d), jnp.float32),
    )(x, w, b)


def _edge_proj(edge_emb_p, p2, w1, bw):
    blk = 4096

    def body(x_ref, p2_ref, w1_ref, bw_ref, epe_ref, ew_ref):
        x = x_ref[...]
        epe_ref[...] = jnp.dot(x, p2_ref[...], preferred_element_type=jnp.float32)
        ew_ref[...] = jnp.dot(x, w1_ref[...],
                              preferred_element_type=jnp.float32) + bw_ref[...]

    return pl.pallas_call(
        body,
        grid=(EP // blk,),
        in_specs=[
            pl.BlockSpec((blk, D_EDGE), lambda i: (i, 0)),
            pl.BlockSpec((D_EDGE, D_MSG), lambda i: (0, 0)),
            pl.BlockSpec((D_EDGE, D_EOUT), lambda i: (0, 0)),
            pl.BlockSpec((1, D_EOUT), lambda i: (0, 0)),
        ],
        out_specs=[
            pl.BlockSpec((blk, D_MSG), lambda i: (i, 0)),
            pl.BlockSpec((blk, D_EOUT), lambda i: (i, 0)),
        ],
        out_shape=[
            jax.ShapeDtypeStruct((EP, D_MSG), jnp.float32),
            jax.ShapeDtypeStruct((EP, D_EOUT), jnp.float32),
        ],
    )(edge_emb_p, p2, w1, bw)


def _node_block(emb_p, sums, cnt, q, bq, w_out, v):
    blk = 128

    def body(emb_ref, sums_ref, cnt_ref, q_ref, bq_ref, w_ref, no_ref, a_ref):
        cnt = jnp.maximum(jnp.sum(cnt_ref[...], axis=0), 1.0)[:, None]
        msg = sums_ref[...] / cnt
        x = jnp.concatenate([emb_ref[...], msg], axis=1)
        no = jnp.dot(x, q_ref[...], preferred_element_type=jnp.float32) + bq_ref[...]
        no_ref[...] = no
        a_ref[...] = jnp.dot(no, w_ref[...], preferred_element_type=jnp.float32)

    return pl.pallas_call(
        body,
        grid=(NP // blk,),
        in_specs=[
            pl.BlockSpec((blk, D_NODE), lambda i: (i, 0)),
            pl.BlockSpec((blk, D_MSG), lambda i, _v=v: (_v * (NP // blk) + i, 0)),
            pl.BlockSpec((NTILE, blk), lambda i, _v=v: (_v, i)),
            pl.BlockSpec((D_NODE + D_MSG, D_OUT), lambda i: (0, 0)),
            pl.BlockSpec((1, D_OUT), lambda i: (0, 0)),
            pl.BlockSpec((D_OUT, D_NODE), lambda i: (0, 0)),
        ],
        out_specs=[
            pl.BlockSpec((blk, D_OUT), lambda i: (i, 0)),
            pl.BlockSpec((blk, D_NODE), lambda i: (i, 0)),
        ],
        out_shape=[
            jax.ShapeDtypeStruct((NP, D_OUT), jnp.float32),
            jax.ShapeDtypeStruct((NP, D_NODE), jnp.float32),
        ],
    )(emb_p, sums, cnt, q, bq, w_out)


def _sc1_body(tbl_h, epe_h, idx_h, sums_h, cnt_h,
              idx_v, gidx_v, rows_v, epe_v, acc, cnt_v, sems):
    c = lax.axis_index("c")
    s = lax.axis_index("s")

    def _zero(r, carry):
        for cc in range(D_MSG // 16):
            rows_v[0, r, pl.ds(cc * 16, 16)] = jnp.zeros((16,), jnp.float32)
        return carry
    lax.fori_loop(0, CH, _zero, 0)

    def _zcnt(j, carry):
        cnt_v[pl.ds(j * 16, 16)] = jnp.zeros((16,), jnp.float32)
        return carry
    lax.fori_loop(0, NP // 16, _zcnt, 0)

    base_r = s * RPT
    for j in range(4):
        pltpu.sync_copy(rows_v.at[0], acc.at[pl.ds(base_r + j * CH, CH)])
    pltpu.sync_copy(rows_v.at[0].at[pl.ds(0, RPT - 4 * CH)],
                    acc.at[pl.ds(base_r + 4 * CH, RPT - 4 * CH)])
    plsc.subcore_barrier()

    def pair(gq, carry):
        pltpu.sync_copy(
            idx_h.at[pl.ds(c * EP + s * EPT + 2 * gq * CH, 2 * CH)], gidx_v)
        for b in range(2):
            kq = 2 * gq + b
            be = s * EPT + kq * CH
            bg = c * EP + be

            @pl.when(kq >= 2)
            def _():
                pltpu.make_async_copy(rows_v.at[b], acc.at[idx_v.at[2 + b]],
                                      sems.at[1 + b]).wait()

            pltpu.sync_copy(idx_h.at[pl.ds(2 * EP + bg, CH)],
                            idx_v.at[2 + b])
            gth = pltpu.async_copy(tbl_h.at[gidx_v.at[pl.ds(b * CH, CH)]],
                                   rows_v.at[b], sems.at[0])

            def _ld(i):
                pltpu.sync_copy(epe_h.at[pl.ds(be + i * EB, EB)], epe_v)

            def _compute(i, _b=b):
                def row(r, rc):
                    for cc in range(D_MSG // 16):
                        csl = pl.ds(cc * 16, 16)
                        rows_v[_b, i * EB + r, csl] = jnp.maximum(
                            rows_v[_b, i * EB + r, csl]
                            + epe_v[r, csl], 0.0)
                    return rc
                lax.fori_loop(0, EB, row, 0)

            _ld(0)
            gth.wait()
            _compute(0)
            _ld(1)
            _compute(1)
            _ld(2)
            _compute(2)
            _ld(3)
            _compute(3)

            pltpu.async_copy(rows_v.at[b], acc.at[idx_v.at[2 + b]],
                             sems.at[1 + b], add=True)
            for i in range(CH // 16):
                idx16 = idx_v[2 + b, pl.ds(i * 16, 16)]
                plsc.addupdate_scatter(cnt_v, [idx16],
                                       jnp.full((16,), 1.0, jnp.float32))
        return carry
    lax.fori_loop(0, NCH1 // 2, pair, 0)

    for b in range(2):
        pltpu.make_async_copy(rows_v.at[b], acc.at[idx_v.at[2 + b]],
                              sems.at[1 + b]).wait()

    plsc.subcore_barrier()
    for j in range(4):
        osl = pl.ds(base_r + j * CH, CH)
        pltpu.sync_copy(acc.at[osl], rows_v.at[0])
        pltpu.sync_copy(rows_v.at[0],
                        sums_h.at[pl.ds(c * NP + base_r + j * CH, CH)])
    vsl = pl.ds(0, RPT - 4 * CH)
    osl = pl.ds(base_r + 4 * CH, RPT - 4 * CH)
    hsl = pl.ds(c * NP + base_r + 4 * CH, RPT - 4 * CH)
    pltpu.sync_copy(acc.at[osl], rows_v.at[0].at[vsl])
    pltpu.sync_copy(rows_v.at[0].at[vsl], sums_h.at[hsl])
    pltpu.sync_copy(cnt_v, cnt_h.at[pl.ds((c * NTILE + s) * NP, NP)])


_sc1 = functools.partial(
    pl.kernel,
    out_type=[
        jax.ShapeDtypeStruct((2 * NP, D_MSG), jnp.float32),
        jax.ShapeDtypeStruct((2 * NTILE * NP,), jnp.float32),
    ],
    mesh=_mesh,
    compiler_params=pltpu.CompilerParams(needs_layout_passes=False),
    scratch_types=[
        pltpu.VMEM((4, CH), jnp.int32),
        pltpu.VMEM((2 * CH,), jnp.int32),
        pltpu.VMEM((2, CH, D_MSG), jnp.float32),
        pltpu.VMEM((EB, D_MSG), jnp.float32),
        pltpu.VMEM_SHARED((NP, D_MSG), jnp.float32),
        pltpu.VMEM((NP,), jnp.float32),
        pltpu.SemaphoreType.DMA((3,)),
    ],
)(_sc1_body)


def _sc2_body(a_h, bv_h, ewf_h, idx_h, eof_h,
              idx_v, arows_v, brows_v, eo_v, sems):
    c = lax.axis_index("c")
    s = lax.axis_index("s")
    base0 = (s * NSC + c) * EPW

    def chunk(kq, b):
        bq = base0 + kq * CH
        osl = pl.ds(bq * D_EOUT, CH * D_EOUT)

        @pl.when(jnp.logical_and(kq >= 2, bq - 2 * CH < E))
        def _():
            pltpu.make_async_copy(eo_v.at[b], eof_h.at[pl.ds(0, CH * D_EOUT)],
                                  sems.at[1 + b]).wait()

        g1 = pltpu.async_copy(a_h.at[idx_v.at[pl.ds(b * CH, CH)]],
                              arows_v.at[b], sems.at[0])
        g2 = pltpu.async_copy(bv_h.at[idx_v.at[pl.ds(2 * CH + b * CH, CH)]],
                              brows_v.at[b], sems.at[0])
        pltpu.sync_copy(ewf_h.at[pl.ds(bq * D_EOUT, CH * D_EOUT)], eo_v.at[b])
        g1.wait()
        g2.wait()

        def row(r, rc, _b=b):
            esl = pl.ds(r * D_EOUT, D_EOUT)
            csl = pl.ds(0, D_EOUT)
            eo_v[_b, esl] = (eo_v[_b, esl] + arows_v[_b, r, csl]
                             + brows_v[_b, r, csl])
            return rc
        lax.fori_loop(0, CH, row, 0)

        @pl.when(bq < E)
        def _():
            pltpu.async_copy(eo_v.at[b], eof_h.at[osl], sems.at[1 + b])

    def pair(gq, carry, nb=2):
        bp = base0 + 2 * gq * CH
        pltpu.sync_copy(idx_h.at[pl.ds(bp, nb * CH)],
                        idx_v.at[pl.ds(0, nb * CH)])
        pltpu.sync_copy(idx_h.at[pl.ds(EP + bp, nb * CH)],
                        idx_v.at[pl.ds(2 * CH, nb * CH)])
        for b in range(nb):
            chunk(2 * gq + b, b)
        return carry
    lax.fori_loop(0, NCH2 // 2, pair, 0)
    pair((NCH2 - 1) // 2, 0, nb=1)
    for b in range(2):
        @pl.when(base0 + (NCH2 - 1 - b) * CH < E)
        def _():
            pltpu.make_async_copy(
                eo_v.at[b], eof_h.at[pl.ds(0, CH * D_EOUT)],
                sems.at[1 + b]).wait()


_sc2 = functools.partial(
    pl.kernel,
    out_type=jax.ShapeDtypeStruct((E * D_EOUT,), jnp.float32),
    mesh=_mesh,
    compiler_params=pltpu.CompilerParams(needs_layout_passes=False),
    scratch_types=[
        pltpu.VMEM((4 * CH,), jnp.int32),
        pltpu.VMEM((2, CH, D_NODE), jnp.float32),
        pltpu.VMEM((2, CH, D_NODE), jnp.float32),
        pltpu.VMEM((2, CH * D_EOUT), jnp.float32),
        pltpu.SemaphoreType.DMA((3,)),
    ],
)(_sc2_body)


def kernel(node_emb, edge_emb, feature_emb, edge_index, P, Q, W, b_P, b_Q, b_W):
    src = edge_index[0]
    dst = edge_index[1]

    p1, p2 = P[:D_NODE], P[D_NODE:]
    w1, w2, w3 = W[:D_EDGE], W[D_EDGE:D_EDGE + D_OUT], W[D_EDGE + D_OUT:]

    pad0 = jnp.zeros((EP - E,), jnp.int32)
    padn = jnp.full((EP - E,), N, jnp.int32)
    src_g = jnp.concatenate([src, pad0])
    dst_g = jnp.concatenate([dst, pad0])
    src_s = jnp.concatenate([src, padn])
    dst_s = jnp.concatenate([dst, padn])
    edge_emb_p = jnp.pad(edge_emb, ((0, EP - E), (0, 0)))

    g = _mm_bias(feature_emb, p1, b_P.reshape(1, -1), 2000)
    gn = _mm_bias(node_emb, p1, b_P.reshape(1, -1), 2000)
    epe, ew = _edge_proj(edge_emb_p, p2, w1, b_W.reshape(1, -1))

    tbl = jnp.concatenate([g, gn], axis=0)
    idx1 = jnp.concatenate([dst_g, src_g + N, src_s, dst_s])
    sums, cnts = _sc1(tbl, epe, idx1)
    cnts2 = cnts.reshape(2 * NTILE, NP)

    node_emb_p = jnp.pad(node_emb, ((0, NP - N), (0, 0)))
    feature_emb_p = jnp.pad(feature_emb, ((0, NP - N), (0, 0)))
    w2p = jnp.pad(w2, ((0, 0), (0, D_NODE - D_EOUT)))
    w3p = jnp.pad(w3, ((0, 0), (0, D_NODE - D_EOUT)))
    node_out, a = _node_block(node_emb_p, sums, cnts2, Q, b_Q.reshape(1, -1),
                              w2p, 0)
    feat_out, bv = _node_block(feature_emb_p, sums, cnts2, Q,
                               b_Q.reshape(1, -1), w3p, 1)

    idx2 = jnp.concatenate([src_g, dst_g])
    eof = _sc2(a, bv, ew.reshape(-1), idx2)
    return node_out[:N], eof.reshape(E, D_EOUT), feat_out[:N]
